# baseline (speedup 1.0000x reference)
import functools

import jax
import jax.numpy as jnp
from jax.experimental import pallas as pl
from jax.experimental.pallas import tpu as pltpu

T, D, H, E, TOPK = 4096, 1024, 4096, 8, 2
TB = 512
HB = 512


def _moe_dense_body(x_ref, wg_ref, w1_ref, w2_ref, bias_ref, out_ref, comb_ref):
    e = pl.program_id(1)
    h = pl.program_id(2)

    @pl.when((e == 0) & (h == 0))
    def _router():
        logits = jnp.dot(x_ref[...], wg_ref[...], preferred_element_type=jnp.float32)
        probs = jax.nn.softmax(logits, axis=-1)
        iota = jax.lax.broadcasted_iota(jnp.int32, (TB, E), 1)
        m0 = jnp.max(probs, axis=1, keepdims=True)
        i0 = jnp.argmax(probs, axis=1)[:, None]
        h0 = iota == i0
        masked = jnp.where(h0, -jnp.inf, probs)
        m1 = jnp.max(masked, axis=1, keepdims=True)
        i1 = jnp.argmax(masked, axis=1)[:, None]
        h1 = iota == i1
        s = m0 + m1 + 1e-9
        comb_ref[...] = jnp.where(h0, m0 / s, 0.0) + jnp.where(h1, m1 / s, 0.0)

    iota_e = jax.lax.broadcasted_iota(jnp.int32, (TB, E), 1)
    c_e = jnp.sum(jnp.where(iota_e == e, comb_ref[...], 0.0), axis=1, keepdims=True)

    hblk = jax.nn.gelu(jnp.dot(x_ref[...], w1_ref[0], preferred_element_type=jnp.float32))
    part = jnp.dot(hblk, w2_ref[0], preferred_element_type=jnp.float32) * c_e

    @pl.when((e == 0) & (h == 0))
    def _init():
        out_ref[...] = part

    @pl.when((e > 0) | (h > 0))
    def _acc():
        out_ref[...] += part

    @pl.when((e == E - 1) & (h == H // HB - 1))
    def _bias():
        out_ref[...] += bias_ref[...]


def kernel(x, Wg, w1, w2, global_bias):
    grid = (T // TB, E, H // HB)
    out = pl.pallas_call(
        _moe_dense_body,
        grid=grid,
        in_specs=[
            pl.BlockSpec((TB, D), lambda t, e, h: (t, 0)),
            pl.BlockSpec((D, E), lambda t, e, h: (0, 0)),
            pl.BlockSpec((1, D, HB), lambda t, e, h: (e, 0, h)),
            pl.BlockSpec((1, HB, D), lambda t, e, h: (e, h, 0)),
            pl.BlockSpec((1, D), lambda t, e, h: (0, 0)),
        ],
        out_specs=pl.BlockSpec((TB, D), lambda t, e, h: (t, 0)),
        out_shape=jax.ShapeDtypeStruct((T, D), jnp.float32),
        scratch_shapes=[pltpu.VMEM((TB, E), jnp.float32)],
        compiler_params=pltpu.CompilerParams(
            dimension_semantics=("arbitrary", "arbitrary", "arbitrary"),
        ),
    )(x, Wg, w1, w2, global_bias.reshape(1, D))
    return out


# speedup vs baseline: 1.4330x; 1.4330x over previous
import functools

import jax
import jax.numpy as jnp
from jax import lax
from jax.experimental import pallas as pl
from jax.experimental.pallas import tpu as pltpu
from jax.experimental.pallas import tpu_sc as plsc

T, D, H, E = 4096, 1024, 4096, 8
NC, NS = 2, 16
NW = NC * NS
TPW = T // NW
MT = 256
M = 2 * T + E * MT
NMT = M // MT
NMT_PAD = 48
HB = 512

_sc_params = pltpu.CompilerParams(needs_layout_passes=False)


def _router_body(x_ref, wg_ref, i0_ref, i1_ref, r0_ref, r1_ref,
                 w0_ref, w1_ref, cnt_ref, carry_ref):
    t = pl.program_id(0)

    @pl.when(t == 0)
    def _():
        carry_ref[...] = jnp.zeros_like(carry_ref)

    logits = jnp.dot(x_ref[...], wg_ref[...], preferred_element_type=jnp.float32)
    probs = jax.nn.softmax(logits, axis=-1)
    iota = lax.broadcasted_iota(jnp.int32, (TPW, E), 1)
    m0 = jnp.max(probs, axis=1, keepdims=True)
    i0 = jnp.argmax(probs, axis=1)[:, None]
    h0 = iota == i0
    masked = jnp.where(h0, -jnp.inf, probs)
    m1 = jnp.max(masked, axis=1, keepdims=True)
    i1 = jnp.argmax(masked, axis=1)[:, None]
    h1 = iota == i1
    s = m0 + m1 + 1e-9

    ind = h0.astype(jnp.float32) + h1.astype(jnp.float32)
    rr = lax.broadcasted_iota(jnp.int32, (TPW, TPW), 0)
    cc = lax.broadcasted_iota(jnp.int32, (TPW, TPW), 1)
    tri = (rr > cc).astype(jnp.float32)
    cumex = jnp.dot(tri, ind, preferred_element_type=jnp.float32)
    tot = cumex + carry_ref[...]
    rank0 = jnp.sum(jnp.where(h0, tot, 0.0), axis=1)
    rank1 = jnp.sum(jnp.where(h1, tot, 0.0), axis=1)

    i0_ref[...] = i0[:, 0].astype(jnp.int32).reshape(1, 1, TPW)
    i1_ref[...] = i1[:, 0].astype(jnp.int32).reshape(1, 1, TPW)
    r0_ref[...] = rank0.astype(jnp.int32).reshape(1, 1, TPW)
    r1_ref[...] = rank1.astype(jnp.int32).reshape(1, 1, TPW)
    w0_ref[...] = (m0[:, 0] / s[:, 0]).reshape(1, 1, TPW)
    w1_ref[...] = (m1[:, 0] / s[:, 0]).reshape(1, 1, TPW)

    carry = carry_ref[...] + jnp.sum(ind, axis=0, keepdims=True)
    carry_ref[...] = carry
    cnt_ref[...] = jnp.concatenate(
        [carry, jnp.zeros_like(carry)], axis=1).astype(jnp.int32).reshape(1, 1, 2 * E)


def _router(x, Wg):
    i3 = lambda sh, dt: jax.ShapeDtypeStruct(sh, dt)
    return pl.pallas_call(
        _router_body,
        grid=(NW,),
        in_specs=[
            pl.BlockSpec((TPW, D), lambda t: (t, 0)),
            pl.BlockSpec((D, E), lambda t: (0, 0)),
        ],
        out_specs=[
            pl.BlockSpec((1, 1, TPW), lambda t: (t, 0, 0)),
            pl.BlockSpec((1, 1, TPW), lambda t: (t, 0, 0)),
            pl.BlockSpec((1, 1, TPW), lambda t: (t, 0, 0)),
            pl.BlockSpec((1, 1, TPW), lambda t: (t, 0, 0)),
            pl.BlockSpec((1, 1, TPW), lambda t: (t, 0, 0)),
            pl.BlockSpec((1, 1, TPW), lambda t: (t, 0, 0)),
            pl.BlockSpec((1, 1, 2 * E), lambda t: (0, 0, 0)),
        ],
        out_shape=[
            i3((NW, 1, TPW), jnp.int32), i3((NW, 1, TPW), jnp.int32),
            i3((NW, 1, TPW), jnp.int32), i3((NW, 1, TPW), jnp.int32),
            i3((NW, 1, TPW), jnp.float32), i3((NW, 1, TPW), jnp.float32),
            i3((1, 1, 2 * E), jnp.int32),
        ],
        scratch_shapes=[pltpu.VMEM((1, E), jnp.float32)],
        compiler_params=pltpu.CompilerParams(dimension_semantics=("arbitrary",)),
    )(x, Wg)


@functools.cache
def _make_dispatch():
    return functools.partial(
        pl.kernel,
        out_type=(
            jax.ShapeDtypeStruct((M, D), jnp.float32),
            jax.ShapeDtypeStruct((NW, 1, TPW), jnp.int32),
            jax.ShapeDtypeStruct((NW, 1, TPW), jnp.int32),
            jax.ShapeDtypeStruct((NMT_PAD,), jnp.int32),
        ),
        mesh=plsc.VectorSubcoreMesh(core_axis_name="c", subcore_axis_name="s"),
        scratch_types=[
        pltpu.VMEM((16,), jnp.int32),
        pltpu.VMEM((16,), jnp.int32),
        pltpu.VMEM((NMT_PAD,), jnp.int32),
        pltpu.VMEM((1, TPW), jnp.int32),
        pltpu.VMEM((1, TPW), jnp.int32),
        pltpu.VMEM((1, TPW), jnp.int32),
        pltpu.VMEM((1, TPW), jnp.int32),
        pltpu.VMEM((1, TPW), jnp.int32),
        pltpu.VMEM((1, TPW), jnp.int32),
        pltpu.VMEM((64, D), jnp.float32),
        pltpu.VMEM((64,), jnp.int32),
        pltpu.VMEM((64,), jnp.int32),
        pltpu.SemaphoreType.DMA,
        ],
        compiler_params=_sc_params,
    )(_dispatch_body)


def _dispatch_body(x_hbm, i0_hbm, i1_hbm, r0_hbm, r1_hbm, cnt_hbm,
              xs_hbm, d0_hbm, d1_hbm, eid_hbm,
              cntv, offv, eidv, i0v, i1v, r0v, r1v, d0v, d1v,
              xrows, diA, diB, sem):
    wid = lax.axis_index("s") * NC + lax.axis_index("c")
    base = wid * TPW

    pltpu.sync_copy(cnt_hbm.at[0, 0], cntv)
    cnt = cntv[...]
    padded = jnp.bitwise_and(cnt + (MT - 1), jnp.int32(-MT))
    incl = plsc.cumsum(padded)
    offv[...] = incl - padded

    pltpu.sync_copy(i0_hbm.at[wid], i0v)
    pltpu.sync_copy(i1_hbm.at[wid], i1v)
    pltpu.sync_copy(r0_hbm.at[wid], r0v)
    pltpu.sync_copy(r1_hbm.at[wid], r1v)

    for j in range(TPW // 16):
        sl = pl.ds(16 * j, 16)
        off0 = plsc.load_gather(offv, [i0v[0, sl]])
        off1 = plsc.load_gather(offv, [i1v[0, sl]])
        d0v[0, sl] = off0 + r0v[0, sl]
        d1v[0, sl] = off1 + r1v[0, sl]
    pltpu.sync_copy(d0v, d0_hbm.at[wid])
    pltpu.sync_copy(d1v, d1_hbm.at[wid])

    @pl.when(wid == 0)
    def _():
        for j in range(NMT_PAD // 16):
            m = lax.iota(jnp.int32, 16) + 16 * j
            pos = m * MT
            acc = jnp.zeros((16,), jnp.int32)
            for e in range(E):
                off_e = plsc.load_gather(offv, [jnp.full((16,), e, jnp.int32)])
                acc = acc + (pos >= off_e).astype(jnp.int32)
            eidv[pl.ds(16 * j, 16)] = acc - 1
        pltpu.sync_copy(eidv, eid_hbm)

    for sub in range(TPW // 64):
        pltpu.sync_copy(x_hbm.at[pl.ds(base + 64 * sub, 64)], xrows)
        for q in range(4):
            diA[pl.ds(16 * q, 16)] = d0v[0, pl.ds(64 * sub + 16 * q, 16)]
            diB[pl.ds(16 * q, 16)] = d1v[0, pl.ds(64 * sub + 16 * q, 16)]
        c0 = pltpu.async_copy(xrows, xs_hbm.at[diA], sem)
        c1 = pltpu.async_copy(xrows, xs_hbm.at[diB], sem)
        c0.wait()
        c1.wait()


def _gemm_body(eid_ref, xs_ref, w1_ref, w2_ref, os_ref):
    h = pl.program_id(1)
    hblk = jax.nn.gelu(jnp.dot(xs_ref[...], w1_ref[0],
                               preferred_element_type=jnp.float32))
    part = jnp.dot(hblk, w2_ref[0], preferred_element_type=jnp.float32)

    @pl.when(h == 0)
    def _():
        os_ref[...] = part

    @pl.when(h > 0)
    def _():
        os_ref[...] += part


def _grouped_gemm(eid, xs, w1, w2):
    grid_spec = pltpu.PrefetchScalarGridSpec(
        num_scalar_prefetch=1,
        grid=(NMT, H // HB),
        in_specs=[
            pl.BlockSpec((MT, D), lambda m, h, eid_ref: (m, 0)),
            pl.BlockSpec((1, D, HB), lambda m, h, eid_ref: (eid_ref[m], 0, h)),
            pl.BlockSpec((1, HB, D), lambda m, h, eid_ref: (eid_ref[m], h, 0)),
        ],
        out_specs=pl.BlockSpec((MT, D), lambda m, h, eid_ref: (m, 0)),
    )
    return pl.pallas_call(
        _gemm_body,
        grid_spec=grid_spec,
        out_shape=jax.ShapeDtypeStruct((M, D), jnp.float32),
        compiler_params=pltpu.CompilerParams(
            dimension_semantics=("arbitrary", "arbitrary")),
    )(eid, xs, w1, w2)


@functools.cache
def _make_combine():
    return functools.partial(
        pl.kernel,
        out_type=jax.ShapeDtypeStruct((T, D), jnp.float32),
        mesh=plsc.VectorSubcoreMesh(core_axis_name="c", subcore_axis_name="s"),
        scratch_types=[
            pltpu.VMEM((1, TPW), jnp.int32),
            pltpu.VMEM((1, TPW), jnp.int32),
            pltpu.VMEM((1, TPW), jnp.float32),
            pltpu.VMEM((1, TPW), jnp.float32),
            pltpu.VMEM((D,), jnp.float32),
            pltpu.VMEM((32,), jnp.int32),
            pltpu.VMEM((32,), jnp.int32),
            pltpu.VMEM((32, D), jnp.float32),
            pltpu.VMEM((32, D), jnp.float32),
            pltpu.VMEM((32, D), jnp.float32),
            pltpu.SemaphoreType.DMA,
        ],
        compiler_params=_sc_params,
    )(_combine_body)


def _combine_body(os_hbm, d0_hbm, d1_hbm, w0_hbm, w1_hbm, bias_hbm, out_hbm,
             d0v, d1v, w0v, w1v, biasv, gA, gB, bufA, bufB, outb, sem):
    wid = lax.axis_index("s") * NC + lax.axis_index("c")
    base = wid * TPW
    pltpu.sync_copy(d0_hbm.at[wid], d0v)
    pltpu.sync_copy(d1_hbm.at[wid], d1v)
    pltpu.sync_copy(w0_hbm.at[wid], w0v)
    pltpu.sync_copy(w1_hbm.at[wid], w1v)
    pltpu.sync_copy(bias_hbm, biasv)
    z16 = jnp.zeros((16,), jnp.int32)
    for sub in range(TPW // 32):
        gA[pl.ds(0, 16)] = d0v[0, pl.ds(32 * sub, 16)]
        gA[pl.ds(16, 16)] = d0v[0, pl.ds(32 * sub + 16, 16)]
        gB[pl.ds(0, 16)] = d1v[0, pl.ds(32 * sub, 16)]
        gB[pl.ds(16, 16)] = d1v[0, pl.ds(32 * sub + 16, 16)]
        cA = pltpu.async_copy(os_hbm.at[gA], bufA, sem)
        cB = pltpu.async_copy(os_hbm.at[gB], bufB, sem)
        cA.wait()
        cB.wait()

        def body(r, carry):
            lane = jnp.full((16,), 32 * sub + r, jnp.int32)
            wa = plsc.load_gather(w0v, [z16, lane])
            wb = plsc.load_gather(w1v, [z16, lane])
            for c in range(D // 16):
                sl = pl.ds(16 * c, 16)
                outb[r, sl] = bufA[r, sl] * wa + bufB[r, sl] * wb + biasv[sl]
            return carry

        lax.fori_loop(0, 32, body, 0)
        pltpu.sync_copy(outb, out_hbm.at[pl.ds(base + 32 * sub, 32)])


def kernel(x, Wg, w1, w2, global_bias):
    i0, i1, r0, r1, w0r, w1r, cnt = _router(x, Wg)
    xs, d0, d1, eid = _make_dispatch()(x, i0, i1, r0, r1, cnt)
    os = _grouped_gemm(eid, xs, w1, w2)
    return _make_combine()(os, d0, d1, w0r, w1r, global_bias)
